# baseline (speedup 1.0000x reference)
import functools

import jax
import jax.numpy as jnp
from jax import lax
from jax.experimental import pallas as pl
from jax.experimental.pallas import tpu as pltpu
from jax.experimental.pallas import tpu_sc as plsc

N = 10000
HALF = N // 2
AR = 5120
PR = AR // 8
SUP = 256
SUB = SUP // 16
NT = 16


def _mesh():
    return plsc.VectorSubcoreMesh(core_axis_name="c", subcore_axis_name="s")


def _loc_pass(dstv, locb, lo, nv):
    def body(v, carry):
        d = dstv[pl.ds(v * 16, 16)]
        loc = d - lo
        m = (loc >= 0) & (loc < HALF)
        locb[pl.ds(v * 16, 16)] = jnp.where(m, loc, HALF + (d & 63))
        return carry
    lax.fori_loop(0, nv, body, 0)


def _make_deg(nch):
    @functools.partial(
        pl.kernel,
        mesh=_mesh(),
        out_type=jax.ShapeDtypeStruct((32 * PR, 128), jnp.float32),
        scratch_types=[
            pltpu.VMEM((SUP,), jnp.int32),
            pltpu.VMEM((SUP,), jnp.int32),
            pltpu.VMEM((PR, 128), jnp.float32),
        ],
    )
    def deg(dst_hbm, zero_hbm, out_hbm, dstv, locb, acc):
        c = lax.axis_index("c")
        s = lax.axis_index("s")
        lo = c * HALF
        wid = c * NT + s
        pltpu.sync_copy(zero_hbm, acc)
        onesv = jnp.full((16,), 1.0, jnp.float32)

        def chunk(j, carry):
            base = s * ((nch // NT) * SUP) + j * SUP
            pltpu.sync_copy(dst_hbm.at[pl.ds(base, SUP)], dstv)
            _loc_pass(dstv, locb, lo, SUP // 16)

            def vec(v, carry2):
                lv = locb[pl.ds(v * 16, 16)]
                for k in range(16):
                    lr = lv[k]
                    pr = lr >> 3
                    po = (lr & 7) * 16
                    acc[pr, pl.ds(po, 16)] = acc[pr, pl.ds(po, 16)] + onesv
                return carry2

            lax.fori_loop(0, SUP // 16, vec, 0)
            return carry

        lax.fori_loop(0, nch // NT, chunk, 0)
        pltpu.sync_copy(acc, out_hbm.at[pl.ds(wid * PR, PR)])

    return deg


def _make_agg(nch):
    @functools.partial(
        pl.kernel,
        mesh=_mesh(),
        out_type=jax.ShapeDtypeStruct((N, 256), jnp.float32),
        scratch_types=[
            pltpu.VMEM((SUB,), jnp.int32),
            pltpu.VMEM((SUP,), jnp.int32),
            pltpu.VMEM((SUP,), jnp.int32),
            pltpu.VMEM((SUB, 256), jnp.float32),
            pltpu.VMEM((SUB // 8, 128), jnp.float32),
            pltpu.VMEM((SUP // 8, 128), jnp.float32),
            pltpu.VMEM((AR // 8, 128), jnp.float32),
            pltpu.VMEM((8, 256), jnp.float32),
            pltpu.VMEM((8, 128), jnp.float32),
            pltpu.VMEM_SHARED((16, SUP // 8, 128), jnp.float32),
            pltpu.VMEM_SHARED((16, 8, 128), jnp.float32),
            pltpu.SemaphoreType.DMA,
        ],
    )
    def agg(u_hbm, src_hbm, dst_hbm, zero_hbm, out_hbm,
            srcv, dstv, locb, rows, pk, cols, acc, wide, tmp,
            stage, asm, gsem):
        c = lax.axis_index("c")
        s = lax.axis_index("s")
        lo = c * HALF
        pltpu.sync_copy(zero_hbm, acc)

        def chunk(j, carry):
            base = j * SUP
            pltpu.sync_copy(src_hbm.at[pl.ds(base + s * SUB, SUB)], srcv)
            pltpu.async_copy(u_hbm.at[srcv], rows, gsem).wait()
            for s2 in range(NT):
                for e in range(SUB):
                    pk[e >> 3, pl.ds((e & 7) * 16, 16)] = (
                        rows[e, pl.ds(s2 * 16, 16)])
                pltpu.sync_copy(pk, stage.at[s2, pl.ds(s * (SUB // 8),
                                                       SUB // 8)])
            pltpu.sync_copy(dst_hbm.at[pl.ds(base, SUP)], dstv)
            _loc_pass(dstv, locb, lo, SUP // 16)
            plsc.subcore_barrier()
            pltpu.sync_copy(stage.at[s], cols)

            def vec(v, carry2):
                lv = locb[pl.ds(v * 16, 16)]
                for k in range(16):
                    lr = lv[k]
                    pr = lr >> 3
                    po = (lr & 7) * 16
                    acc[pr, pl.ds(po, 16)] = (
                        acc[pr, pl.ds(po, 16)]
                        + cols[v * 2 + (k >> 3), pl.ds((k & 7) * 16, 16)])
                return carry2

            lax.fori_loop(0, SUP // 16, vec, 0)
            plsc.subcore_barrier()
            return carry

        lax.fori_loop(0, nch, chunk, 0)

        def emit(r0):
            for s2 in range(NT):
                pltpu.sync_copy(asm.at[s2], tmp)
                for rr in range(8):
                    wide[rr, pl.ds(s2 * 16, 16)] = tmp[s, pl.ds(rr * 16, 16)]
            pltpu.sync_copy(wide, out_hbm.at[pl.ds(lo + r0 + s * 8, 8)])

        def out_round(r, carry):
            pltpu.sync_copy(acc.at[pl.ds(r * 8, 8)], asm.at[s])
            plsc.subcore_barrier()

            @pl.when(s < 8)
            def _():
                emit(r * 64)

            plsc.subcore_barrier()
            return carry

        lax.fori_loop(0, HALF // 64, out_round, 0)
        pltpu.sync_copy(acc.at[pl.ds(624, 8)], asm.at[s])
        plsc.subcore_barrier()

        @pl.when(s == 0)
        def _():
            for s2 in range(NT):
                pltpu.sync_copy(asm.at[s2], tmp)
                for rr in range(8):
                    wide[rr, pl.ds(s2 * 16, 16)] = tmp[0, pl.ds(rr * 16, 16)]
            pltpu.sync_copy(wide, out_hbm.at[pl.ds(lo + 4992, 8)])

    return agg


BM = 1000


def _hist_body(parts_ref, dp_ref):
    hist = jnp.sum(parts_ref[...], axis=(0, 1))
    dp_ref[...] = lax.rsqrt(hist + 1.0)[None, :, :]


def _scale_body(dinv_ref, x_ref, o_ref):
    o_ref[...] = x_ref[...] * dinv_ref[...]


def _l1_body(dinv_ref, agg_ref, xp_ref, w_ref, b_ref, oa_ref, ob_ref):
    dinv = dinv_ref[...]
    z = (agg_ref[...] + xp_ref[...]) * dinv
    w = w_ref[...]
    h = jnp.dot(z, w, preferred_element_type=jnp.float32) + b_ref[...]
    h = jnp.maximum(h, 0.0) * dinv
    oa_ref[...] = h[:, :256]
    ob_ref[...] = h[:, 256:]


def _l2_body(dinv_ref, aa_ref, ab_ref, ha_ref, hb_ref, w_ref, b_ref, o_ref):
    dinv = dinv_ref[...]
    za = (aa_ref[...] + ha_ref[...]) * dinv
    zb = (ab_ref[...] + hb_ref[...]) * dinv
    w = w_ref[...]
    o_ref[...] = (jnp.dot(za, w[:256, :], preferred_element_type=jnp.float32)
                  + jnp.dot(zb, w[256:, :], preferred_element_type=jnp.float32)
                  + b_ref[...])


def _row_spec(cols):
    return pl.BlockSpec((BM, cols), lambda i: (i, 0))


def _full_spec(r, cols):
    return pl.BlockSpec((r, cols), lambda i: (0, 0))


def kernel(x, edge_index, W1, b1, W2, b2):
    src = edge_index[0].astype(jnp.int32)
    dst = edge_index[1].astype(jnp.int32)
    e = src.shape[0]
    nch = -(-e // SUP)
    nch = NT * (-(-nch // NT))
    epad = nch * SUP
    src_p = jnp.concatenate([src, jnp.zeros((epad - e,), jnp.int32)])
    dst_p = jnp.concatenate([dst, jnp.full((epad - e,), -1, jnp.int32)])

    zpack = jnp.zeros((PR, 128), jnp.float32)
    zacc = jnp.zeros((AR // 8, 128), jnp.float32)

    parts = _make_deg(nch)(dst_p, zpack)
    parts = parts.reshape(2, 16, PR, 128)

    dp = pl.pallas_call(
        _hist_body,
        grid=(20,),
        in_specs=[pl.BlockSpec((1, 16, PR // 10, 128),
                               lambda i: (i // 10, 0, i % 10, 0))],
        out_specs=pl.BlockSpec((1, PR // 10, 128),
                               lambda i: (i // 10, i % 10, 0)),
        out_shape=jax.ShapeDtypeStruct((2, PR, 128), jnp.float32),
    )(parts)
    dinv = dp.reshape(2, PR, 8, 16)[:, :, :, 0].reshape(2, AR)[:, :HALF]
    dinv = dinv.reshape(N, 1)

    grid = N // BM
    xp = pl.pallas_call(
        _scale_body,
        grid=(grid,),
        in_specs=[_row_spec(1), _row_spec(256)],
        out_specs=_row_spec(256),
        out_shape=jax.ShapeDtypeStruct((N, 256), jnp.float32),
    )(dinv, x)

    agg = _make_agg(nch)
    agg1 = agg(xp, src_p, dst_p, zacc)

    h1a, h1b = pl.pallas_call(
        _l1_body,
        grid=(grid,),
        in_specs=[_row_spec(1), _row_spec(256), _row_spec(256),
                  _full_spec(256, 512), _full_spec(1, 512)],
        out_specs=[_row_spec(256), _row_spec(256)],
        out_shape=[jax.ShapeDtypeStruct((N, 256), jnp.float32),
                   jax.ShapeDtypeStruct((N, 256), jnp.float32)],
    )(dinv, agg1, xp, W1, b1.reshape(1, 512))

    agg2a = agg(h1a, src_p, dst_p, zacc)
    agg2b = agg(h1b, src_p, dst_p, zacc)

    out = pl.pallas_call(
        _l2_body,
        grid=(grid,),
        in_specs=[_row_spec(1), _row_spec(256), _row_spec(256),
                  _row_spec(256), _row_spec(256),
                  _full_spec(512, 512), _full_spec(1, 512)],
        out_specs=_row_spec(512),
        out_shape=jax.ShapeDtypeStruct((N, 512), jnp.float32),
    )(dinv, agg2a, agg2b, h1a, h1b, W2, b2.reshape(1, 512))
    return out


# speedup vs baseline: 1.3135x; 1.3135x over previous
import functools

import jax
import jax.numpy as jnp
from jax import lax
from jax.experimental import pallas as pl
from jax.experimental.pallas import tpu as pltpu
from jax.experimental.pallas import tpu_sc as plsc

N = 10000
HALF = N // 2
AR = 5120
PR = AR // 8
SUP = 512
SUB = SUP // 16
NT = 16


def _mesh():
    return plsc.VectorSubcoreMesh(core_axis_name="c", subcore_axis_name="s")


def _loc_pass(dstv, locb, lo, nv):
    def body(v, carry):
        d = dstv[pl.ds(v * 16, 16)]
        loc = d - lo
        m = (loc >= 0) & (loc < HALF)
        locb[pl.ds(v * 16, 16)] = jnp.where(m, loc, HALF + (d & 63))
        return carry
    lax.fori_loop(0, nv, body, 0)


def _make_deg(nch):
    @functools.partial(
        pl.kernel,
        mesh=_mesh(),
        out_type=jax.ShapeDtypeStruct((32 * PR, 128), jnp.float32),
        scratch_types=[
            pltpu.VMEM((SUP,), jnp.int32),
            pltpu.VMEM((SUP,), jnp.int32),
            pltpu.VMEM((PR, 128), jnp.float32),
        ],
    )
    def deg(dst_hbm, zero_hbm, out_hbm, dstv, locb, acc):
        c = lax.axis_index("c")
        s = lax.axis_index("s")
        lo = c * HALF
        wid = c * NT + s
        pltpu.sync_copy(zero_hbm, acc)
        onesv = jnp.full((16,), 1.0, jnp.float32)

        def chunk(j, carry):
            base = s * ((nch // NT) * SUP) + j * SUP
            pltpu.sync_copy(dst_hbm.at[pl.ds(base, SUP)], dstv)
            _loc_pass(dstv, locb, lo, SUP // 16)

            def vec(v, carry2):
                lv = locb[pl.ds(v * 16, 16)]
                for k in range(16):
                    lr = lv[k]
                    pr = lr >> 3
                    po = (lr & 7) * 16
                    acc[pr, pl.ds(po, 16)] = acc[pr, pl.ds(po, 16)] + onesv
                return carry2

            lax.fori_loop(0, SUP // 16, vec, 0)
            return carry

        lax.fori_loop(0, nch // NT, chunk, 0)
        pltpu.sync_copy(acc, out_hbm.at[pl.ds(wid * PR, PR)])

    return deg


def _make_agg(nch):
    @functools.partial(
        pl.kernel,
        mesh=_mesh(),
        out_type=jax.ShapeDtypeStruct((N, 256), jnp.float32),
        scratch_types=[
            pltpu.VMEM((SUB,), jnp.int32),
            pltpu.VMEM((SUP,), jnp.int32),
            pltpu.VMEM((SUP,), jnp.int32),
            pltpu.VMEM((SUB, 256), jnp.float32),
            pltpu.VMEM((SUB // 8, 128), jnp.float32),
            pltpu.VMEM((SUP // 8, 128), jnp.float32),
            pltpu.VMEM((AR // 8, 128), jnp.float32),
            pltpu.VMEM((8, 256), jnp.float32),
            pltpu.VMEM((8, 128), jnp.float32),
            pltpu.VMEM_SHARED((16, SUP // 8, 128), jnp.float32),
            pltpu.VMEM_SHARED((16, 8, 128), jnp.float32),
            pltpu.SemaphoreType.DMA,
        ],
    )
    def agg(u_hbm, src_hbm, dst_hbm, zero_hbm, out_hbm,
            srcv, dstv, locb, rows, pk, cols, acc, wide, tmp,
            stage, asm, gsem):
        c = lax.axis_index("c")
        s = lax.axis_index("s")
        lo = c * HALF
        pltpu.sync_copy(zero_hbm, acc)

        def chunk(j, carry):
            base = j * SUP
            pltpu.sync_copy(src_hbm.at[pl.ds(base + s * SUB, SUB)], srcv)
            pltpu.async_copy(u_hbm.at[srcv], rows, gsem).wait()
            for s2 in range(NT):
                for e in range(SUB):
                    pk[e >> 3, pl.ds((e & 7) * 16, 16)] = (
                        rows[e, pl.ds(s2 * 16, 16)])
                pltpu.sync_copy(pk, stage.at[s2, pl.ds(s * (SUB // 8),
                                                       SUB // 8)])
            pltpu.sync_copy(dst_hbm.at[pl.ds(base, SUP)], dstv)
            _loc_pass(dstv, locb, lo, SUP // 16)
            plsc.subcore_barrier()
            pltpu.sync_copy(stage.at[s], cols)

            def vec(v, carry2):
                lv = locb[pl.ds(v * 16, 16)]
                for k in range(16):
                    lr = lv[k]
                    pr = lr >> 3
                    po = (lr & 7) * 16
                    acc[pr, pl.ds(po, 16)] = (
                        acc[pr, pl.ds(po, 16)]
                        + cols[v * 2 + (k >> 3), pl.ds((k & 7) * 16, 16)])
                return carry2

            lax.fori_loop(0, SUP // 16, vec, 0)
            plsc.subcore_barrier()
            return carry

        lax.fori_loop(0, nch, chunk, 0)

        def emit(r0):
            for s2 in range(NT):
                pltpu.sync_copy(asm.at[s2], tmp)
                for rr in range(8):
                    wide[rr, pl.ds(s2 * 16, 16)] = tmp[s, pl.ds(rr * 16, 16)]
            pltpu.sync_copy(wide, out_hbm.at[pl.ds(lo + r0 + s * 8, 8)])

        def out_round(r, carry):
            pltpu.sync_copy(acc.at[pl.ds(r * 8, 8)], asm.at[s])
            plsc.subcore_barrier()

            @pl.when(s < 8)
            def _():
                emit(r * 64)

            plsc.subcore_barrier()
            return carry

        lax.fori_loop(0, HALF // 64, out_round, 0)
        pltpu.sync_copy(acc.at[pl.ds(624, 8)], asm.at[s])
        plsc.subcore_barrier()

        @pl.when(s == 0)
        def _():
            for s2 in range(NT):
                pltpu.sync_copy(asm.at[s2], tmp)
                for rr in range(8):
                    wide[rr, pl.ds(s2 * 16, 16)] = tmp[0, pl.ds(rr * 16, 16)]
            pltpu.sync_copy(wide, out_hbm.at[pl.ds(lo + 4992, 8)])

    return agg


BM = 1000


def _hist_body(parts_ref, dp_ref):
    hist = jnp.sum(parts_ref[...], axis=(0, 1))
    dp_ref[...] = lax.rsqrt(hist + 1.0)[None, :, :]


def _scale_body(dinv_ref, x_ref, o_ref):
    o_ref[...] = x_ref[...] * dinv_ref[...]


def _l1_body(dinv_ref, agg_ref, xp_ref, w_ref, b_ref, oa_ref, ob_ref):
    dinv = dinv_ref[...]
    z = (agg_ref[...] + xp_ref[...]) * dinv
    w = w_ref[...]
    h = jnp.dot(z, w, preferred_element_type=jnp.float32) + b_ref[...]
    h = jnp.maximum(h, 0.0) * dinv
    oa_ref[...] = h[:, :256]
    ob_ref[...] = h[:, 256:]


def _l2_body(dinv_ref, aa_ref, ab_ref, ha_ref, hb_ref, w_ref, b_ref, o_ref):
    dinv = dinv_ref[...]
    za = (aa_ref[...] + ha_ref[...]) * dinv
    zb = (ab_ref[...] + hb_ref[...]) * dinv
    w = w_ref[...]
    o_ref[...] = (jnp.dot(za, w[:256, :], preferred_element_type=jnp.float32)
                  + jnp.dot(zb, w[256:, :], preferred_element_type=jnp.float32)
                  + b_ref[...])


def _row_spec(cols):
    return pl.BlockSpec((BM, cols), lambda i: (i, 0))


def _full_spec(r, cols):
    return pl.BlockSpec((r, cols), lambda i: (0, 0))


def kernel(x, edge_index, W1, b1, W2, b2):
    src = edge_index[0].astype(jnp.int32)
    dst = edge_index[1].astype(jnp.int32)
    e = src.shape[0]
    nch = -(-e // SUP)
    nch = NT * (-(-nch // NT))
    epad = nch * SUP
    src_p = jnp.concatenate([src, jnp.zeros((epad - e,), jnp.int32)])
    dst_p = jnp.concatenate([dst, jnp.full((epad - e,), -1, jnp.int32)])

    zpack = jnp.zeros((PR, 128), jnp.float32)
    zacc = jnp.zeros((AR // 8, 128), jnp.float32)

    parts = _make_deg(nch)(dst_p, zpack)
    parts = parts.reshape(2, 16, PR, 128)

    dp = pl.pallas_call(
        _hist_body,
        grid=(20,),
        in_specs=[pl.BlockSpec((1, 16, PR // 10, 128),
                               lambda i: (i // 10, 0, i % 10, 0))],
        out_specs=pl.BlockSpec((1, PR // 10, 128),
                               lambda i: (i // 10, i % 10, 0)),
        out_shape=jax.ShapeDtypeStruct((2, PR, 128), jnp.float32),
    )(parts)
    dinv = dp.reshape(2, PR, 8, 16)[:, :, :, 0].reshape(2, AR)[:, :HALF]
    dinv = dinv.reshape(N, 1)

    grid = N // BM
    xp = pl.pallas_call(
        _scale_body,
        grid=(grid,),
        in_specs=[_row_spec(1), _row_spec(256)],
        out_specs=_row_spec(256),
        out_shape=jax.ShapeDtypeStruct((N, 256), jnp.float32),
    )(dinv, x)

    agg = _make_agg(nch)
    agg1 = agg(xp, src_p, dst_p, zacc)

    h1a, h1b = pl.pallas_call(
        _l1_body,
        grid=(grid,),
        in_specs=[_row_spec(1), _row_spec(256), _row_spec(256),
                  _full_spec(256, 512), _full_spec(1, 512)],
        out_specs=[_row_spec(256), _row_spec(256)],
        out_shape=[jax.ShapeDtypeStruct((N, 256), jnp.float32),
                   jax.ShapeDtypeStruct((N, 256), jnp.float32)],
    )(dinv, agg1, xp, W1, b1.reshape(1, 512))

    agg2a = agg(h1a, src_p, dst_p, zacc)
    agg2b = agg(h1b, src_p, dst_p, zacc)

    out = pl.pallas_call(
        _l2_body,
        grid=(grid,),
        in_specs=[_row_spec(1), _row_spec(256), _row_spec(256),
                  _row_spec(256), _row_spec(256),
                  _full_spec(512, 512), _full_spec(1, 512)],
        out_specs=_row_spec(512),
        out_shape=jax.ShapeDtypeStruct((N, 512), jnp.float32),
    )(dinv, agg2a, agg2b, h1a, h1b, W2, b2.reshape(1, 512))
    return out


# speedup vs baseline: 1.5730x; 1.1976x over previous
import functools

import jax
import jax.numpy as jnp
from jax import lax
from jax.experimental import pallas as pl
from jax.experimental.pallas import tpu as pltpu
from jax.experimental.pallas import tpu_sc as plsc

N = 10000
HALF = N // 2
AR = 5120
PR = AR // 8
SUP = 512
SUB = SUP // 16
NT = 16


def _mesh():
    return plsc.VectorSubcoreMesh(core_axis_name="c", subcore_axis_name="s")


def _loc_pass(dstv, locb, lo, nv):
    def body(v, carry):
        d = dstv[pl.ds(v * 16, 16)]
        loc = d - lo
        m = (loc >= 0) & (loc < HALF)
        locb[pl.ds(v * 16, 16)] = jnp.where(m, loc, HALF + (d & 63))
        return carry
    lax.fori_loop(0, nv, body, 0)


def _make_deg(nch):
    @functools.partial(
        pl.kernel,
        mesh=_mesh(),
        out_type=jax.ShapeDtypeStruct((32 * PR, 128), jnp.float32),
        scratch_types=[
            pltpu.VMEM((SUP,), jnp.int32),
            pltpu.VMEM((SUP,), jnp.int32),
            pltpu.VMEM((PR, 128), jnp.float32),
        ],
    )
    def deg(dst_hbm, zero_hbm, out_hbm, dstv, locb, acc):
        c = lax.axis_index("c")
        s = lax.axis_index("s")
        lo = c * HALF
        wid = c * NT + s
        pltpu.sync_copy(zero_hbm, acc)
        onesv = jnp.full((16,), 1.0, jnp.float32)

        def chunk(j, carry):
            base = s * ((nch // NT) * SUP) + j * SUP
            pltpu.sync_copy(dst_hbm.at[pl.ds(base, SUP)], dstv)
            _loc_pass(dstv, locb, lo, SUP // 16)

            def vec(v, carry2):
                lv = locb[pl.ds(v * 16, 16)]
                for k in range(16):
                    lr = lv[k]
                    pr = lr >> 3
                    po = (lr & 7) * 16
                    acc[pr, pl.ds(po, 16)] = acc[pr, pl.ds(po, 16)] + onesv
                return carry2

            lax.fori_loop(0, SUP // 16, vec, 0)
            return carry

        lax.fori_loop(0, nch // NT, chunk, 0)
        pltpu.sync_copy(acc, out_hbm.at[pl.ds(wid * PR, PR)])

    return deg


def _make_agg(nch):
    @functools.partial(
        pl.kernel,
        mesh=_mesh(),
        out_type=jax.ShapeDtypeStruct((N, 256), jnp.float32),
        scratch_types=[
            pltpu.VMEM((SUB,), jnp.int32),
            pltpu.VMEM((SUP,), jnp.int32),
            pltpu.VMEM((SUP,), jnp.int32),
            pltpu.VMEM((SUB, 256), jnp.float32),
            pltpu.VMEM((16, SUB // 8, 128), jnp.float32),
            pltpu.VMEM((SUP // 8, 128), jnp.float32),
            pltpu.VMEM((AR // 8, 128), jnp.float32),
            pltpu.VMEM((8, 256), jnp.float32),
            pltpu.VMEM((8, 128), jnp.float32),
            pltpu.VMEM_SHARED((16, SUP // 8, 128), jnp.float32),
            pltpu.VMEM_SHARED((16, 8, 128), jnp.float32),
            pltpu.SemaphoreType.DMA,
        ],
    )
    def agg(u_hbm, src_hbm, dst_hbm, zero_hbm, out_hbm,
            srcv, dstv, locb, rows, pk, cols, acc, wide, tmp,
            stage, asm, gsem):
        c = lax.axis_index("c")
        s = lax.axis_index("s")
        lo = c * HALF
        pltpu.sync_copy(zero_hbm, acc)

        def chunk(j, carry):
            base = j * SUP
            pltpu.sync_copy(src_hbm.at[pl.ds(base + s * SUB, SUB)], srcv)
            gcp = pltpu.async_copy(u_hbm.at[srcv], rows, gsem)
            pltpu.sync_copy(dst_hbm.at[pl.ds(base, SUP)], dstv)
            _loc_pass(dstv, locb, lo, SUP // 16)
            gcp.wait()
            for s2 in range(NT):
                for e in range(SUB):
                    pk[s2, e >> 3, pl.ds((e & 7) * 16, 16)] = (
                        rows[e, pl.ds(s2 * 16, 16)])
            cps = [pltpu.async_copy(
                       pk.at[s2],
                       stage.at[s2, pl.ds(s * (SUB // 8), SUB // 8)], gsem)
                   for s2 in range(NT)]
            for cp in cps:
                cp.wait()
            plsc.subcore_barrier()
            pltpu.sync_copy(stage.at[s], cols)

            def vec(v, carry2):
                lv = locb[pl.ds(v * 16, 16)]
                for k in range(16):
                    lr = lv[k]
                    pr = lr >> 3
                    po = (lr & 7) * 16
                    acc[pr, pl.ds(po, 16)] = (
                        acc[pr, pl.ds(po, 16)]
                        + cols[v * 2 + (k >> 3), pl.ds((k & 7) * 16, 16)])
                return carry2

            lax.fori_loop(0, SUP // 16, vec, 0)
            plsc.subcore_barrier()
            return carry

        lax.fori_loop(0, nch, chunk, 0)

        def emit(r0):
            for s2 in range(NT):
                pltpu.sync_copy(asm.at[s2], tmp)
                for rr in range(8):
                    wide[rr, pl.ds(s2 * 16, 16)] = tmp[s, pl.ds(rr * 16, 16)]
            pltpu.sync_copy(wide, out_hbm.at[pl.ds(lo + r0 + s * 8, 8)])

        def out_round(r, carry):
            pltpu.sync_copy(acc.at[pl.ds(r * 8, 8)], asm.at[s])
            plsc.subcore_barrier()

            @pl.when(s < 8)
            def _():
                emit(r * 64)

            plsc.subcore_barrier()
            return carry

        lax.fori_loop(0, HALF // 64, out_round, 0)
        pltpu.sync_copy(acc.at[pl.ds(624, 8)], asm.at[s])
        plsc.subcore_barrier()

        @pl.when(s == 0)
        def _():
            for s2 in range(NT):
                pltpu.sync_copy(asm.at[s2], tmp)
                for rr in range(8):
                    wide[rr, pl.ds(s2 * 16, 16)] = tmp[0, pl.ds(rr * 16, 16)]
            pltpu.sync_copy(wide, out_hbm.at[pl.ds(lo + 4992, 8)])

    return agg


BM = 1000


def _hist_body(parts_ref, dp_ref):
    hist = jnp.sum(parts_ref[...], axis=(0, 1))
    dp_ref[...] = lax.rsqrt(hist + 1.0)[None, :, :]


def _scale_body(dinv_ref, x_ref, o_ref):
    o_ref[...] = x_ref[...] * dinv_ref[...]


def _l1_body(dinv_ref, agg_ref, xp_ref, w_ref, b_ref, oa_ref, ob_ref):
    dinv = dinv_ref[...]
    z = (agg_ref[...] + xp_ref[...]) * dinv
    w = w_ref[...]
    h = jnp.dot(z, w, preferred_element_type=jnp.float32) + b_ref[...]
    h = jnp.maximum(h, 0.0) * dinv
    oa_ref[...] = h[:, :256]
    ob_ref[...] = h[:, 256:]


def _l2_body(dinv_ref, aa_ref, ab_ref, ha_ref, hb_ref, w_ref, b_ref, o_ref):
    dinv = dinv_ref[...]
    za = (aa_ref[...] + ha_ref[...]) * dinv
    zb = (ab_ref[...] + hb_ref[...]) * dinv
    w = w_ref[...]
    o_ref[...] = (jnp.dot(za, w[:256, :], preferred_element_type=jnp.float32)
                  + jnp.dot(zb, w[256:, :], preferred_element_type=jnp.float32)
                  + b_ref[...])


def _row_spec(cols):
    return pl.BlockSpec((BM, cols), lambda i: (i, 0))


def _full_spec(r, cols):
    return pl.BlockSpec((r, cols), lambda i: (0, 0))


def kernel(x, edge_index, W1, b1, W2, b2):
    src = edge_index[0].astype(jnp.int32)
    dst = edge_index[1].astype(jnp.int32)
    e = src.shape[0]
    nch = -(-e // SUP)
    nch = NT * (-(-nch // NT))
    epad = nch * SUP
    src_p = jnp.concatenate([src, jnp.zeros((epad - e,), jnp.int32)])
    dst_p = jnp.concatenate([dst, jnp.full((epad - e,), -1, jnp.int32)])

    zpack = jnp.zeros((PR, 128), jnp.float32)
    zacc = jnp.zeros((AR // 8, 128), jnp.float32)

    parts = _make_deg(nch)(dst_p, zpack)
    parts = parts.reshape(2, 16, PR, 128)

    dp = pl.pallas_call(
        _hist_body,
        grid=(20,),
        in_specs=[pl.BlockSpec((1, 16, PR // 10, 128),
                               lambda i: (i // 10, 0, i % 10, 0))],
        out_specs=pl.BlockSpec((1, PR // 10, 128),
                               lambda i: (i // 10, i % 10, 0)),
        out_shape=jax.ShapeDtypeStruct((2, PR, 128), jnp.float32),
    )(parts)
    dinv = dp.reshape(2, PR, 8, 16)[:, :, :, 0].reshape(2, AR)[:, :HALF]
    dinv = dinv.reshape(N, 1)

    grid = N // BM
    xp = pl.pallas_call(
        _scale_body,
        grid=(grid,),
        in_specs=[_row_spec(1), _row_spec(256)],
        out_specs=_row_spec(256),
        out_shape=jax.ShapeDtypeStruct((N, 256), jnp.float32),
    )(dinv, x)

    agg = _make_agg(nch)
    agg1 = agg(xp, src_p, dst_p, zacc)

    h1a, h1b = pl.pallas_call(
        _l1_body,
        grid=(grid,),
        in_specs=[_row_spec(1), _row_spec(256), _row_spec(256),
                  _full_spec(256, 512), _full_spec(1, 512)],
        out_specs=[_row_spec(256), _row_spec(256)],
        out_shape=[jax.ShapeDtypeStruct((N, 256), jnp.float32),
                   jax.ShapeDtypeStruct((N, 256), jnp.float32)],
    )(dinv, agg1, xp, W1, b1.reshape(1, 512))

    agg2a = agg(h1a, src_p, dst_p, zacc)
    agg2b = agg(h1b, src_p, dst_p, zacc)

    out = pl.pallas_call(
        _l2_body,
        grid=(grid,),
        in_specs=[_row_spec(1), _row_spec(256), _row_spec(256),
                  _row_spec(256), _row_spec(256),
                  _full_spec(512, 512), _full_spec(1, 512)],
        out_specs=_row_spec(512),
        out_shape=jax.ShapeDtypeStruct((N, 512), jnp.float32),
    )(dinv, agg2a, agg2b, h1a, h1b, W2, b2.reshape(1, 512))
    return out


# speedup vs baseline: 1.5730x; 1.0000x over previous
import functools

import jax
import jax.numpy as jnp
from jax import lax
from jax.experimental import pallas as pl
from jax.experimental.pallas import tpu as pltpu
from jax.experimental.pallas import tpu_sc as plsc

N = 10000
HALF = N // 2
AR = 5120
PR = AR // 8
SUP = 512
SUB = SUP // 16
NT = 16


def _mesh():
    return plsc.VectorSubcoreMesh(core_axis_name="c", subcore_axis_name="s")


def _loc_pass(dstv, locb, lo, nv):
    def body(v, carry):
        d = dstv[pl.ds(v * 16, 16)]
        loc = d - lo
        m = (loc >= 0) & (loc < HALF)
        locb[pl.ds(v * 16, 16)] = jnp.where(m, loc, HALF + (d & 63))
        return carry
    lax.fori_loop(0, nv, body, 0)


def _make_deg(nch):
    @functools.partial(
        pl.kernel,
        mesh=_mesh(),
        out_type=jax.ShapeDtypeStruct((32 * PR, 128), jnp.float32),
        scratch_types=[
            pltpu.VMEM((SUP,), jnp.int32),
            pltpu.VMEM((SUP,), jnp.int32),
            pltpu.VMEM((PR, 128), jnp.float32),
        ],
    )
    def deg(dst_hbm, zero_hbm, out_hbm, dstv, locb, acc):
        c = lax.axis_index("c")
        s = lax.axis_index("s")
        lo = c * HALF
        wid = c * NT + s
        pltpu.sync_copy(zero_hbm, acc)
        onesv = jnp.full((16,), 1.0, jnp.float32)

        def chunk(j, carry):
            base = s * ((nch // NT) * SUP) + j * SUP
            pltpu.sync_copy(dst_hbm.at[pl.ds(base, SUP)], dstv)
            _loc_pass(dstv, locb, lo, SUP // 16)

            def vec(v, carry2):
                lv = locb[pl.ds(v * 16, 16)]
                for k in range(16):
                    lr = lv[k]
                    pr = lr >> 3
                    po = (lr & 7) * 16
                    acc[pr, pl.ds(po, 16)] = acc[pr, pl.ds(po, 16)] + onesv
                return carry2

            lax.fori_loop(0, SUP // 16, vec, 0)
            return carry

        lax.fori_loop(0, nch // NT, chunk, 0)
        pltpu.sync_copy(acc, out_hbm.at[pl.ds(wid * PR, PR)])

    return deg


def _make_agg(nch):
    @functools.partial(
        pl.kernel,
        mesh=_mesh(),
        out_type=jax.ShapeDtypeStruct((N, 256), jnp.float32),
        scratch_types=[
            pltpu.VMEM((SUB,), jnp.int32),
            pltpu.VMEM((SUP,), jnp.int32),
            pltpu.VMEM((SUP,), jnp.int32),
            pltpu.VMEM((SUB, 256), jnp.float32),
            pltpu.VMEM((16, SUB // 8, 128), jnp.float32),
            pltpu.VMEM((SUP // 8, 128), jnp.float32),
            pltpu.VMEM((AR // 8, 128), jnp.float32),
            pltpu.VMEM((8, 256), jnp.float32),
            pltpu.VMEM((8, 128), jnp.float32),
            pltpu.VMEM_SHARED((16, SUP // 8, 128), jnp.float32),
            pltpu.VMEM_SHARED((16, 8, 128), jnp.float32),
            pltpu.SemaphoreType.DMA,
        ],
    )
    def agg(u_hbm, src_hbm, dst_hbm, zero_hbm, out_hbm,
            srcv, dstv, locb, rows, pk, cols, acc, wide, tmp,
            stage, asm, gsem):
        c = lax.axis_index("c")
        s = lax.axis_index("s")
        lo = c * HALF
        pltpu.sync_copy(zero_hbm, acc)

        def chunk(j, carry):
            base = j * SUP
            pltpu.sync_copy(src_hbm.at[pl.ds(base + s * SUB, SUB)], srcv)
            gcp = pltpu.async_copy(u_hbm.at[srcv], rows, gsem)
            pltpu.sync_copy(dst_hbm.at[pl.ds(base, SUP)], dstv)
            _loc_pass(dstv, locb, lo, SUP // 16)
            gcp.wait()
            for s2 in range(NT):
                for e in range(SUB):
                    pk[s2, e >> 3, pl.ds((e & 7) * 16, 16)] = (
                        rows[e, pl.ds(s2 * 16, 16)])
            cps = [pltpu.async_copy(
                       pk.at[s2],
                       stage.at[s2, pl.ds(s * (SUB // 8), SUB // 8)], gsem)
                   for s2 in range(NT)]
            for cp in cps:
                cp.wait()
            plsc.subcore_barrier()
            pltpu.sync_copy(stage.at[s], cols)
            plsc.subcore_barrier()

            def vec(v, carry2):
                lv = locb[pl.ds(v * 16, 16)]
                for k in range(16):
                    lr = lv[k]
                    pr = lr >> 3
                    po = (lr & 7) * 16
                    acc[pr, pl.ds(po, 16)] = (
                        acc[pr, pl.ds(po, 16)]
                        + cols[v * 2 + (k >> 3), pl.ds((k & 7) * 16, 16)])
                return carry2

            lax.fori_loop(0, SUP // 16, vec, 0)
            return carry

        lax.fori_loop(0, nch, chunk, 0)

        def emit(r0):
            for s2 in range(NT):
                pltpu.sync_copy(asm.at[s2], tmp)
                for rr in range(8):
                    wide[rr, pl.ds(s2 * 16, 16)] = tmp[s, pl.ds(rr * 16, 16)]
            pltpu.sync_copy(wide, out_hbm.at[pl.ds(lo + r0 + s * 8, 8)])

        def out_round(r, carry):
            pltpu.sync_copy(acc.at[pl.ds(r * 8, 8)], asm.at[s])
            plsc.subcore_barrier()

            @pl.when(s < 8)
            def _():
                emit(r * 64)

            plsc.subcore_barrier()
            return carry

        lax.fori_loop(0, HALF // 64, out_round, 0)
        pltpu.sync_copy(acc.at[pl.ds(624, 8)], asm.at[s])
        plsc.subcore_barrier()

        @pl.when(s == 0)
        def _():
            for s2 in range(NT):
                pltpu.sync_copy(asm.at[s2], tmp)
                for rr in range(8):
                    wide[rr, pl.ds(s2 * 16, 16)] = tmp[0, pl.ds(rr * 16, 16)]
            pltpu.sync_copy(wide, out_hbm.at[pl.ds(lo + 4992, 8)])

    return agg


BM = 1000


def _hist_body(parts_ref, dp_ref):
    hist = jnp.sum(parts_ref[...], axis=(0, 1))
    dp_ref[...] = lax.rsqrt(hist + 1.0)[None, :, :]


def _scale_body(dinv_ref, x_ref, o_ref):
    o_ref[...] = x_ref[...] * dinv_ref[...]


def _l1_body(dinv_ref, agg_ref, xp_ref, w_ref, b_ref, oa_ref, ob_ref):
    dinv = dinv_ref[...]
    z = (agg_ref[...] + xp_ref[...]) * dinv
    w = w_ref[...]
    h = jnp.dot(z, w, preferred_element_type=jnp.float32) + b_ref[...]
    h = jnp.maximum(h, 0.0) * dinv
    oa_ref[...] = h[:, :256]
    ob_ref[...] = h[:, 256:]


def _l2_body(dinv_ref, aa_ref, ab_ref, ha_ref, hb_ref, w_ref, b_ref, o_ref):
    dinv = dinv_ref[...]
    za = (aa_ref[...] + ha_ref[...]) * dinv
    zb = (ab_ref[...] + hb_ref[...]) * dinv
    w = w_ref[...]
    o_ref[...] = (jnp.dot(za, w[:256, :], preferred_element_type=jnp.float32)
                  + jnp.dot(zb, w[256:, :], preferred_element_type=jnp.float32)
                  + b_ref[...])


def _row_spec(cols):
    return pl.BlockSpec((BM, cols), lambda i: (i, 0))


def _full_spec(r, cols):
    return pl.BlockSpec((r, cols), lambda i: (0, 0))


def kernel(x, edge_index, W1, b1, W2, b2):
    src = edge_index[0].astype(jnp.int32)
    dst = edge_index[1].astype(jnp.int32)
    e = src.shape[0]
    nch = -(-e // SUP)
    nch = NT * (-(-nch // NT))
    epad = nch * SUP
    src_p = jnp.concatenate([src, jnp.zeros((epad - e,), jnp.int32)])
    dst_p = jnp.concatenate([dst, jnp.full((epad - e,), -1, jnp.int32)])

    zpack = jnp.zeros((PR, 128), jnp.float32)
    zacc = jnp.zeros((AR // 8, 128), jnp.float32)

    parts = _make_deg(nch)(dst_p, zpack)
    parts = parts.reshape(2, 16, PR, 128)

    dp = pl.pallas_call(
        _hist_body,
        grid=(20,),
        in_specs=[pl.BlockSpec((1, 16, PR // 10, 128),
                               lambda i: (i // 10, 0, i % 10, 0))],
        out_specs=pl.BlockSpec((1, PR // 10, 128),
                               lambda i: (i // 10, i % 10, 0)),
        out_shape=jax.ShapeDtypeStruct((2, PR, 128), jnp.float32),
    )(parts)
    dinv = dp.reshape(2, PR, 8, 16)[:, :, :, 0].reshape(2, AR)[:, :HALF]
    dinv = dinv.reshape(N, 1)

    grid = N // BM
    xp = pl.pallas_call(
        _scale_body,
        grid=(grid,),
        in_specs=[_row_spec(1), _row_spec(256)],
        out_specs=_row_spec(256),
        out_shape=jax.ShapeDtypeStruct((N, 256), jnp.float32),
    )(dinv, x)

    agg = _make_agg(nch)
    agg1 = agg(xp, src_p, dst_p, zacc)

    h1a, h1b = pl.pallas_call(
        _l1_body,
        grid=(grid,),
        in_specs=[_row_spec(1), _row_spec(256), _row_spec(256),
                  _full_spec(256, 512), _full_spec(1, 512)],
        out_specs=[_row_spec(256), _row_spec(256)],
        out_shape=[jax.ShapeDtypeStruct((N, 256), jnp.float32),
                   jax.ShapeDtypeStruct((N, 256), jnp.float32)],
    )(dinv, agg1, xp, W1, b1.reshape(1, 512))

    agg2a = agg(h1a, src_p, dst_p, zacc)
    agg2b = agg(h1b, src_p, dst_p, zacc)

    out = pl.pallas_call(
        _l2_body,
        grid=(grid,),
        in_specs=[_row_spec(1), _row_spec(256), _row_spec(256),
                  _row_spec(256), _row_spec(256),
                  _full_spec(512, 512), _full_spec(1, 512)],
        out_specs=_row_spec(512),
        out_shape=jax.ShapeDtypeStruct((N, 512), jnp.float32),
    )(dinv, agg2a, agg2b, h1a, h1b, W2, b2.reshape(1, 512))
    return out
